# baseline (speedup 1.0000x reference)
import functools

import jax
import jax.numpy as jnp
from jax import lax
from jax.experimental import pallas as pl
from jax.experimental.pallas import tpu as pltpu
from jax.experimental.pallas import tpu_sc as plsc

N = 10000
E = 320000
DF = 128
D = 64
NCORE = 2
NSUB = 16
NW = NCORE * NSUB
EPW = E // NW
K = 80
NCHUNK = EPW // K
NP = 10240
RPS = NP // NSUB

_mesh = plsc.VectorSubcoreMesh(core_axis_name="c", subcore_axis_name="s")


@functools.partial(
    pl.kernel,
    out_type=jax.ShapeDtypeStruct((NCORE, NP, D), jnp.float32),
    mesh=_mesh,
    scratch_types=[
        pltpu.VMEM((NCHUNK, K), jnp.int32),
        pltpu.VMEM((NCHUNK, K), jnp.int32),
        [pltpu.VMEM((K, D), jnp.float32)] * 8,
        pltpu.VMEM_SHARED((NP, D), jnp.float32),
        [pltpu.SemaphoreType.DMA] * 8,
        [pltpu.SemaphoreType.DMA] * 8,
    ],
    compiler_params=pltpu.CompilerParams(use_tc_tiling_on_sc=False, skip_device_barrier=True),
)
def _sc_scatter(g_hbm, e_hbm, zero_hbm, out_hbm,
                src_v, dst_v, rows, acc_sh, gsem, ssem):
    c = lax.axis_index("c")
    s = lax.axis_index("s")
    pltpu.sync_copy(zero_hbm.at[pl.ds(s * RPS, RPS)],
                    acc_sh.at[pl.ds(s * RPS, RPS)])
    pltpu.sync_copy(e_hbm.at[0, c, s], src_v)
    pltpu.sync_copy(e_hbm.at[1, c, s], dst_v)
    plsc.subcore_barrier()

    def gather(j, b):
        pltpu.async_copy(g_hbm.at[src_v.at[j]], rows[b], gsem[b])

    def gather_wait(j, b):
        pltpu.make_async_copy(g_hbm.at[src_v.at[j]], rows[b], gsem[b]).wait()

    def scatter(j, b):
        pltpu.async_copy(rows[b], acc_sh.at[dst_v.at[j]], ssem[b], add=True)

    def scatter_wait(j, b):
        pltpu.make_async_copy(rows[b], acc_sh.at[dst_v.at[j]], ssem[b]).wait()

    NB = 8
    RING = NCHUNK - NCHUNK % NB
    for b in range(NB):
        gather(b, b)

    def body(i, carry):
        for b in range(NB):
            j = NB * i + b
            gather_wait(j, b)
            scatter(j, b)
        for b in range(NB):
            jn = NB * i + NB + b

            @pl.when(jn < RING)
            def _():
                scatter_wait(jn - NB, b)
                gather(jn, b)
        return carry

    lax.fori_loop(0, RING // NB, body, 0)
    for b in range(NB):
        scatter_wait(RING - NB + b, b)
    for t, j in enumerate(range(RING, NCHUNK)):
        gather(j, t)
    for t, j in enumerate(range(RING, NCHUNK)):
        gather_wait(j, t)
        scatter(j, t)
    for t, j in enumerate(range(RING, NCHUNK)):
        scatter_wait(j, t)
    plsc.subcore_barrier()
    pltpu.sync_copy(acc_sh.at[pl.ds(s * RPS, RPS)],
                    out_hbm.at[c, pl.ds(s * RPS, RPS)])


@functools.partial(
    pl.kernel,
    out_type=jax.ShapeDtypeStruct((NCORE, NP, 8), jnp.float32),
    mesh=_mesh,
    scratch_types=[
        pltpu.VMEM((NCHUNK, K), jnp.int32),
        pltpu.VMEM((K, 8), jnp.float32),
        pltpu.VMEM_SHARED((NP, 8), jnp.float32),
        pltpu.SemaphoreType.DMA,
    ],
    compiler_params=pltpu.CompilerParams(use_tc_tiling_on_sc=False, skip_device_barrier=True),
)
def _sc_degree(e_hbm, ones_hbm, zero_hbm, out_hbm, dst_v, ones_v, acc_sh, sem):
    c = lax.axis_index("c")
    s = lax.axis_index("s")
    pltpu.sync_copy(zero_hbm.at[pl.ds(s * RPS, RPS)],
                    acc_sh.at[pl.ds(s * RPS, RPS)])
    pltpu.sync_copy(e_hbm.at[1, c, s], dst_v)
    pltpu.sync_copy(ones_hbm, ones_v)
    plsc.subcore_barrier()

    for j0 in range(8):
        pltpu.async_copy(ones_v, acc_sh.at[dst_v.at[j0]], sem, add=True)

    def body(j, carry):
        pltpu.make_async_copy(ones_v, acc_sh.at[dst_v.at[j]], sem).wait()
        pltpu.async_copy(ones_v, acc_sh.at[dst_v.at[j + 8]], sem, add=True)
        return carry

    lax.fori_loop(0, NCHUNK - 8, body, 0)
    for j0 in range(NCHUNK - 8, NCHUNK):
        pltpu.make_async_copy(ones_v, acc_sh.at[dst_v.at[j0]], sem).wait()
    plsc.subcore_barrier()
    pltpu.sync_copy(acc_sh.at[pl.ds(s * RPS, RPS)],
                    out_hbm.at[c, pl.ds(s * RPS, RPS)])


_BLK = 1000
_NBLK = N // _BLK


def _dinv(degp_ref):
    deg = degp_ref[0, :, 0:1] + degp_ref[1, :, 0:1] + 1.0
    return lax.rsqrt(deg)


def _tc_mm_body(x_ref, w1_ref, h_ref):
    h_ref[...] = jnp.dot(x_ref[...], w1_ref[...],
                         preferred_element_type=jnp.float32)


def _tc_mm(x, w1):
    return pl.pallas_call(
        _tc_mm_body,
        grid=(_NBLK,),
        in_specs=[
            pl.BlockSpec((_BLK, DF), lambda i: (i, 0)),
            pl.BlockSpec((DF, D), lambda i: (0, 0)),
        ],
        out_specs=pl.BlockSpec((_BLK, D), lambda i: (i, 0)),
        out_shape=jax.ShapeDtypeStruct((N, D), jnp.float32),
    )(x, w1)


def _tc_scale_body(degp_ref, h_ref, g_ref):
    g_ref[...] = h_ref[...] * _dinv(degp_ref)


def _tc_scale(degp, h):
    return pl.pallas_call(
        _tc_scale_body,
        grid=(_NBLK,),
        in_specs=[
            pl.BlockSpec((NCORE, _BLK, 8), lambda i: (0, i, 0)),
            pl.BlockSpec((_BLK, D), lambda i: (i, 0)),
        ],
        out_specs=pl.BlockSpec((_BLK, D), lambda i: (i, 0)),
        out_shape=jax.ShapeDtypeStruct((N, D), jnp.float32),
    )(degp, h)


def _tc_mid_body(p_ref, g_ref, degp_ref, b1_ref, w2_ref, g2_ref):
    dinv = _dinv(degp_ref)
    z = jax.nn.relu((p_ref[0] + p_ref[1] + g_ref[...]) * dinv + b1_ref[...])
    h2 = jnp.dot(z, w2_ref[...], preferred_element_type=jnp.float32)
    g2_ref[...] = h2 * dinv


def _tc_mid(p, g1, degp, b1, w2):
    return pl.pallas_call(
        _tc_mid_body,
        grid=(_NBLK,),
        in_specs=[
            pl.BlockSpec((NCORE, _BLK, D), lambda i: (0, i, 0)),
            pl.BlockSpec((_BLK, D), lambda i: (i, 0)),
            pl.BlockSpec((NCORE, _BLK, 8), lambda i: (0, i, 0)),
            pl.BlockSpec((1, D), lambda i: (0, 0)),
            pl.BlockSpec((D, D), lambda i: (0, 0)),
        ],
        out_specs=pl.BlockSpec((_BLK, D), lambda i: (i, 0)),
        out_shape=jax.ShapeDtypeStruct((N, D), jnp.float32),
    )(p, g1, degp, b1.reshape(1, D), w2)


def _tc_final_body(p_ref, g_ref, degp_ref, b2_ref, wa_ref, ba_ref,
                   wb_ref, bb_ref, h_ref, o3_ref):
    h = jax.nn.relu((p_ref[0] + p_ref[1] + g_ref[...]) * _dinv(degp_ref)
                    + b2_ref[...])
    h_ref[...] = h
    t = jax.nn.relu(jnp.dot(h, wa_ref[...], preferred_element_type=jnp.float32)
                    + ba_ref[...])
    o3_ref[...] = jax.nn.sigmoid(
        jnp.dot(t, wb_ref[...], preferred_element_type=jnp.float32)
        + bb_ref[...])


def _tc_final(p, g2, degp, b2, wa, ba, wb, bb):
    return pl.pallas_call(
        _tc_final_body,
        grid=(_NBLK,),
        in_specs=[
            pl.BlockSpec((NCORE, _BLK, D), lambda i: (0, i, 0)),
            pl.BlockSpec((_BLK, D), lambda i: (i, 0)),
            pl.BlockSpec((NCORE, _BLK, 8), lambda i: (0, i, 0)),
            pl.BlockSpec((1, D), lambda i: (0, 0)),
            pl.BlockSpec((D, 96), lambda i: (0, 0)),
            pl.BlockSpec((1, 96), lambda i: (0, 0)),
            pl.BlockSpec((96, 8), lambda i: (0, 0)),
            pl.BlockSpec((1, 8), lambda i: (0, 0)),
        ],
        out_specs=[
            pl.BlockSpec((_BLK, D), lambda i: (i, 0)),
            pl.BlockSpec((_BLK, 8), lambda i: (i, 0)),
        ],
        out_shape=[
            jax.ShapeDtypeStruct((N, D), jnp.float32),
            jax.ShapeDtypeStruct((N, 8), jnp.float32),
        ],
    )(p, g2, degp, b2.reshape(1, D), wa, ba, wb, bb)


def kernel(x, edge_index, W1, b1, W2, b2, dWa, dba, dWb, dbb,
           iWa, iba, iWb, ibb, cWa, cba, cWb, cbb):
    edges = edge_index.astype(jnp.int32).reshape(2, NCORE, NSUB, NCHUNK, K)
    zero64 = jnp.zeros((NP, D), jnp.float32)
    zero8 = jnp.zeros((NP, 8), jnp.float32)

    h1 = _tc_mm(x, W1)
    degp = _sc_degree(edges, jnp.ones((K, 8), jnp.float32), zero8)
    g1 = _tc_scale(degp, h1)
    p1 = _sc_scatter(g1, edges, zero64)
    g2 = _tc_mid(p1, g1, degp, b1, W2)
    p2 = _sc_scatter(g2, edges, zero64)

    wa = jnp.concatenate([dWa, iWa, cWa], axis=1)
    ba = jnp.concatenate([dba, iba, cba]).reshape(1, 96)
    wb = jnp.zeros((96, 8), jnp.float32)
    wb = wb.at[0:32, 0].set(dWb[:, 0]).at[32:64, 1].set(iWb[:, 0])
    wb = wb.at[64:96, 2].set(cWb[:, 0])
    bb = jnp.zeros((1, 8), jnp.float32)
    bb = bb.at[0, 0].set(dbb[0]).at[0, 1].set(ibb[0]).at[0, 2].set(cbb[0])

    h, o3 = _tc_final(p2, g2, degp, b2, wa, ba, wb, bb)
    return h, o3[:, 0:1], o3[:, 1:2], o3[:, 2:3]


# speedup vs baseline: 1.0077x; 1.0077x over previous
import functools

import jax
import jax.numpy as jnp
from jax import lax
from jax.experimental import pallas as pl
from jax.experimental.pallas import tpu as pltpu
from jax.experimental.pallas import tpu_sc as plsc

N = 10000
E = 320000
DF = 128
D = 64
NCORE = 2
NSUB = 16
NW = NCORE * NSUB
EPW = E // NW
K = 80
NCHUNK = EPW // K
NP = 10240
RPS = NP // NSUB

_mesh = plsc.VectorSubcoreMesh(core_axis_name="c", subcore_axis_name="s")


@functools.partial(
    pl.kernel,
    out_type=jax.ShapeDtypeStruct((NCORE, NP, D), jnp.float32),
    mesh=_mesh,
    scratch_types=[
        pltpu.VMEM((NCHUNK, K), jnp.int32),
        pltpu.VMEM((NCHUNK, K), jnp.int32),
        [pltpu.VMEM((K, D), jnp.float32)] * 8,
        pltpu.VMEM_SHARED((NP, D), jnp.float32),
        [pltpu.SemaphoreType.DMA] * 8,
        [pltpu.SemaphoreType.DMA] * 8,
    ],
    compiler_params=pltpu.CompilerParams(use_tc_tiling_on_sc=False, skip_device_barrier=True),
)
def _sc_scatter(g_hbm, e_hbm, zero_hbm, out_hbm,
                src_v, dst_v, rows, acc_sh, gsem, ssem):
    c = lax.axis_index("c")
    s = lax.axis_index("s")
    pltpu.sync_copy(zero_hbm.at[pl.ds(s * RPS, RPS)],
                    acc_sh.at[pl.ds(s * RPS, RPS)])
    pltpu.sync_copy(e_hbm.at[0, c, s], src_v)
    pltpu.sync_copy(e_hbm.at[1, c, s], dst_v)
    plsc.subcore_barrier()

    def gather(j, b):
        pltpu.async_copy(g_hbm.at[src_v.at[j]], rows[b], gsem[b])

    def gather_wait(j, b):
        pltpu.make_async_copy(g_hbm.at[src_v.at[j]], rows[b], gsem[b]).wait()

    def scatter(j, b):
        pltpu.async_copy(rows[b], acc_sh.at[dst_v.at[j]], ssem[b], add=True)

    def scatter_wait(j, b):
        pltpu.make_async_copy(rows[b], acc_sh.at[dst_v.at[j]], ssem[b]).wait()

    NB = 8
    RING = NCHUNK - NCHUNK % NB
    for b in range(NB):
        gather(b, b)

    def body(i, carry):
        for b in range(NB):
            j = NB * i + b
            gather_wait(j, b)
            scatter(j, b)
        for b in range(NB):
            jn = NB * i + NB + b

            @pl.when(jn < RING)
            def _():
                scatter_wait(jn - NB, b)
                gather(jn, b)
        return carry

    lax.fori_loop(0, RING // NB, body, 0)
    for b in range(NB):
        scatter_wait(RING - NB + b, b)
    for t, j in enumerate(range(RING, NCHUNK)):
        gather(j, t)
    for t, j in enumerate(range(RING, NCHUNK)):
        gather_wait(j, t)
        scatter(j, t)
    for t, j in enumerate(range(RING, NCHUNK)):
        scatter_wait(j, t)
    plsc.subcore_barrier()
    pltpu.sync_copy(acc_sh.at[pl.ds(s * RPS, RPS)],
                    out_hbm.at[c, pl.ds(s * RPS, RPS)])


@functools.partial(
    pl.kernel,
    out_type=jax.ShapeDtypeStruct((NCORE, NP, 8), jnp.float32),
    mesh=_mesh,
    scratch_types=[
        pltpu.VMEM((NCHUNK, K), jnp.int32),
        pltpu.VMEM((K, 8), jnp.float32),
        pltpu.VMEM_SHARED((NP, 8), jnp.float32),
        pltpu.SemaphoreType.DMA,
    ],
    compiler_params=pltpu.CompilerParams(use_tc_tiling_on_sc=False, skip_device_barrier=True),
)
def _sc_degree(e_hbm, ones_hbm, zero_hbm, out_hbm, dst_v, ones_v, acc_sh, sem):
    c = lax.axis_index("c")
    s = lax.axis_index("s")
    pltpu.sync_copy(zero_hbm.at[pl.ds(s * RPS, RPS)],
                    acc_sh.at[pl.ds(s * RPS, RPS)])
    pltpu.sync_copy(e_hbm.at[1, c, s], dst_v)
    pltpu.sync_copy(ones_hbm, ones_v)
    plsc.subcore_barrier()

    for j0 in range(8):
        pltpu.async_copy(ones_v, acc_sh.at[dst_v.at[j0]], sem, add=True)

    def body(j, carry):
        pltpu.make_async_copy(ones_v, acc_sh.at[dst_v.at[j]], sem).wait()
        pltpu.async_copy(ones_v, acc_sh.at[dst_v.at[j + 8]], sem, add=True)
        return carry

    lax.fori_loop(0, NCHUNK - 8, body, 0)
    for j0 in range(NCHUNK - 8, NCHUNK):
        pltpu.make_async_copy(ones_v, acc_sh.at[dst_v.at[j0]], sem).wait()
    plsc.subcore_barrier()
    pltpu.sync_copy(acc_sh.at[pl.ds(s * RPS, RPS)],
                    out_hbm.at[c, pl.ds(s * RPS, RPS)])


_BLK = 1000
_NBLK = N // _BLK


def _dinv(degp_ref):
    deg = degp_ref[0, :, 0:1] + degp_ref[1, :, 0:1] + 1.0
    return lax.rsqrt(deg)


def _tc_pre_body(degp_ref, x_ref, w1_ref, g_ref):
    h = jnp.dot(x_ref[...], w1_ref[...], preferred_element_type=jnp.float32)
    g_ref[...] = h * _dinv(degp_ref)


def _tc_pre(degp, x, w1):
    return pl.pallas_call(
        _tc_pre_body,
        grid=(_NBLK,),
        in_specs=[
            pl.BlockSpec((NCORE, _BLK, 8), lambda i: (0, i, 0)),
            pl.BlockSpec((_BLK, DF), lambda i: (i, 0)),
            pl.BlockSpec((DF, D), lambda i: (0, 0)),
        ],
        out_specs=pl.BlockSpec((_BLK, D), lambda i: (i, 0)),
        out_shape=jax.ShapeDtypeStruct((N, D), jnp.float32),
    )(degp, x, w1)


def _tc_mid_body(p_ref, g_ref, degp_ref, b1_ref, w2_ref, g2_ref):
    dinv = _dinv(degp_ref)
    z = jax.nn.relu((p_ref[0] + p_ref[1] + g_ref[...]) * dinv + b1_ref[...])
    h2 = jnp.dot(z, w2_ref[...], preferred_element_type=jnp.float32)
    g2_ref[...] = h2 * dinv


def _tc_mid(p, g1, degp, b1, w2):
    return pl.pallas_call(
        _tc_mid_body,
        grid=(_NBLK,),
        in_specs=[
            pl.BlockSpec((NCORE, _BLK, D), lambda i: (0, i, 0)),
            pl.BlockSpec((_BLK, D), lambda i: (i, 0)),
            pl.BlockSpec((NCORE, _BLK, 8), lambda i: (0, i, 0)),
            pl.BlockSpec((1, D), lambda i: (0, 0)),
            pl.BlockSpec((D, D), lambda i: (0, 0)),
        ],
        out_specs=pl.BlockSpec((_BLK, D), lambda i: (i, 0)),
        out_shape=jax.ShapeDtypeStruct((N, D), jnp.float32),
    )(p, g1, degp, b1.reshape(1, D), w2)


def _tc_final_body(p_ref, g_ref, degp_ref, b2_ref, wa_ref, ba_ref,
                   wb_ref, bb_ref, h_ref, o3_ref):
    h = jax.nn.relu((p_ref[0] + p_ref[1] + g_ref[...]) * _dinv(degp_ref)
                    + b2_ref[...])
    h_ref[...] = h
    t = jax.nn.relu(jnp.dot(h, wa_ref[...], preferred_element_type=jnp.float32)
                    + ba_ref[...])
    o3_ref[...] = jax.nn.sigmoid(
        jnp.dot(t, wb_ref[...], preferred_element_type=jnp.float32)
        + bb_ref[...])


def _tc_final(p, g2, degp, b2, wa, ba, wb, bb):
    return pl.pallas_call(
        _tc_final_body,
        grid=(_NBLK,),
        in_specs=[
            pl.BlockSpec((NCORE, _BLK, D), lambda i: (0, i, 0)),
            pl.BlockSpec((_BLK, D), lambda i: (i, 0)),
            pl.BlockSpec((NCORE, _BLK, 8), lambda i: (0, i, 0)),
            pl.BlockSpec((1, D), lambda i: (0, 0)),
            pl.BlockSpec((D, 96), lambda i: (0, 0)),
            pl.BlockSpec((1, 96), lambda i: (0, 0)),
            pl.BlockSpec((96, 8), lambda i: (0, 0)),
            pl.BlockSpec((1, 8), lambda i: (0, 0)),
        ],
        out_specs=[
            pl.BlockSpec((_BLK, D), lambda i: (i, 0)),
            pl.BlockSpec((_BLK, 8), lambda i: (i, 0)),
        ],
        out_shape=[
            jax.ShapeDtypeStruct((N, D), jnp.float32),
            jax.ShapeDtypeStruct((N, 8), jnp.float32),
        ],
    )(p, g2, degp, b2.reshape(1, D), wa, ba, wb, bb)


def kernel(x, edge_index, W1, b1, W2, b2, dWa, dba, dWb, dbb,
           iWa, iba, iWb, ibb, cWa, cba, cWb, cbb):
    edges = edge_index.astype(jnp.int32).reshape(2, NCORE, NSUB, NCHUNK, K)
    zero64 = jnp.zeros((NP, D), jnp.float32)
    zero8 = jnp.zeros((NP, 8), jnp.float32)

    degp = _sc_degree(edges, jnp.ones((K, 8), jnp.float32), zero8)
    g1 = _tc_pre(degp, x, W1)
    p1 = _sc_scatter(g1, edges, zero64)
    g2 = _tc_mid(p1, g1, degp, b1, W2)
    p2 = _sc_scatter(g2, edges, zero64)

    wa = jnp.concatenate([dWa, iWa, cWa], axis=1)
    ba = jnp.concatenate([dba, iba, cba]).reshape(1, 96)
    wb = jnp.zeros((96, 8), jnp.float32)
    wb = wb.at[0:32, 0].set(dWb[:, 0]).at[32:64, 1].set(iWb[:, 0])
    wb = wb.at[64:96, 2].set(cWb[:, 0])
    bb = jnp.zeros((1, 8), jnp.float32)
    bb = bb.at[0, 0].set(dbb[0]).at[0, 1].set(ibb[0]).at[0, 2].set(cbb[0])

    h, o3 = _tc_final(p2, g2, degp, b2, wa, ba, wb, bb)
    return h, o3[:, 0:1], o3[:, 1:2], o3[:, 2:3]
